# Initial kernel scaffold; baseline (speedup 1.0000x reference)
import jax
import jax.numpy as jnp
from jax.experimental import pallas as pl


def kernel(query_latent, query_context, keys, contexts, ruls, sohs, k):
    raise NotImplementedError("write your pallas kernel here")


# speedup vs baseline: 1.5144x; 1.5144x over previous
import functools

import jax
import jax.numpy as jnp
from jax import lax
from jax.experimental import pallas as pl
from jax.experimental.pallas import tpu as pltpu

PHYSICS_W = 0.7
CONTEXT_W = 0.3
TOPK = 5
_NEG_INF = float("-inf")


def _score_topk_body(q_ref, qc_ref, keys_ref, ctx_ref, out_s_ref, out_i_ref,
                     run_s, run_i, *, kt, nkt):
    j = pl.program_id(1)

    @pl.when(j == 0)
    def _init():
        run_s[...] = jnp.full(run_s.shape, _NEG_INF, jnp.float32)
        run_i[...] = jnp.zeros(run_i.shape, jnp.int32)

    q = q_ref[...]
    qn = q / jnp.maximum(jnp.sqrt(jnp.sum(q * q, axis=1, keepdims=True)), 1e-12)
    qc = qc_ref[...]
    qcn = qc / jnp.maximum(jnp.sqrt(jnp.sum(qc * qc, axis=1, keepdims=True)), 1e-12)

    kt_rows = keys_ref[...]
    ktn = kt_rows / jnp.maximum(
        jnp.sqrt(jnp.sum(kt_rows * kt_rows, axis=1, keepdims=True)), 1e-12)
    ct_rows = ctx_ref[...]
    ctn = ct_rows / jnp.maximum(
        jnp.sqrt(jnp.sum(ct_rows * ct_rows, axis=1, keepdims=True)), 1e-12)

    dims = (((1,), (1,)), ((), ()))
    s = PHYSICS_W * lax.dot_general(qn, ktn, dims,
                                    preferred_element_type=jnp.float32)
    s = s + CONTEXT_W * lax.dot_general(qcn, ctn, dims,
                                        preferred_element_type=jnp.float32)

    qt = s.shape[0]
    gcol = j * kt + lax.broadcasted_iota(jnp.int32, (qt, kt), 1)
    c = jnp.concatenate([run_s[...], s], axis=1)
    ci = jnp.concatenate([run_i[...], gcol], axis=1)
    w = c.shape[1]
    pos = lax.broadcasted_iota(jnp.int32, (qt, w), 1)

    new_s = []
    new_i = []
    for _ in range(TOPK):
        m = jnp.max(c, axis=1, keepdims=True)
        am = jnp.min(jnp.where(c == m, pos, w), axis=1, keepdims=True)
        sel = pos == am
        new_s.append(m)
        new_i.append(jnp.sum(jnp.where(sel, ci, 0), axis=1, keepdims=True))
        c = jnp.where(sel, _NEG_INF, c)

    run_s[:, 0:TOPK] = jnp.concatenate(new_s, axis=1)
    run_i[:, 0:TOPK] = jnp.concatenate(new_i, axis=1)

    @pl.when(j == nkt - 1)
    def _emit():
        out_s_ref[...] = run_s[:, 0:TOPK]
        out_i_ref[...] = run_i[:, 0:TOPK]


def _score_topk(query_latent, query_context, keys, contexts, *, qt=256, kt=2000):
    q, d = query_latent.shape
    k, _ = keys.shape
    p = query_context.shape[1]
    assert q % qt == 0 and k % kt == 0
    nqt = q // qt
    nkt = k // kt

    grid = (nqt, nkt)
    body = functools.partial(_score_topk_body, kt=kt, nkt=nkt)
    out_s, out_i = pl.pallas_call(
        body,
        grid=grid,
        in_specs=[
            pl.BlockSpec((qt, d), lambda i, j: (i, 0)),
            pl.BlockSpec((qt, p), lambda i, j: (i, 0)),
            pl.BlockSpec((kt, d), lambda i, j: (j, 0)),
            pl.BlockSpec((kt, p), lambda i, j: (j, 0)),
        ],
        out_specs=[
            pl.BlockSpec((qt, TOPK), lambda i, j: (i, 0)),
            pl.BlockSpec((qt, TOPK), lambda i, j: (i, 0)),
        ],
        out_shape=[
            jax.ShapeDtypeStruct((q, TOPK), jnp.float32),
            jax.ShapeDtypeStruct((q, TOPK), jnp.int32),
        ],
        scratch_shapes=[
            pltpu.VMEM((qt, 8), jnp.float32),
            pltpu.VMEM((qt, 8), jnp.int32),
        ],
    )(query_latent, query_context, keys, contexts)
    return out_s, out_i


def kernel(query_latent, query_context, keys, contexts, ruls, sohs, k):
    topk_scores, topk_idx = _score_topk(query_latent, query_context, keys,
                                        contexts)
    retrieved_keys = jnp.take(keys, topk_idx, axis=0)
    retrieved_ruls = jnp.take(ruls, topk_idx, axis=0)
    retrieved_sohs = jnp.take(sohs, topk_idx, axis=0)
    return retrieved_keys, retrieved_ruls, retrieved_sohs, topk_scores


# speedup vs baseline: 3.2729x; 2.1612x over previous
import functools

import jax
import jax.numpy as jnp
from jax import lax
from jax.experimental import pallas as pl
from jax.experimental.pallas import tpu as pltpu

PHYSICS_W = 0.7
CONTEXT_W = 0.3
TOPK = 5
_NEG_INF = float("-inf")
_I32_MAX = jnp.iinfo(jnp.int32).max


def _score_topk_body(q_ref, qc_ref, keys_ref, ctx_ref, out_s_ref, out_i_ref,
                     ls_ref, is_ref, *, kt, nkt, ktotal, chunk):
    j = pl.program_id(0)
    nch = kt // chunk

    @pl.when(j == 0)
    def _init():
        ls_ref[...] = jnp.full(ls_ref.shape, _NEG_INF, jnp.float32)
        is_ref[...] = jnp.zeros(is_ref.shape, jnp.int32)

    q = q_ref[...]
    qn = q * (PHYSICS_W / jnp.maximum(
        jnp.sqrt(jnp.sum(q * q, axis=1, keepdims=True)), 1e-12))
    qc = qc_ref[...]
    qcn = qc * (CONTEXT_W / jnp.maximum(
        jnp.sqrt(jnp.sum(qc * qc, axis=1, keepdims=True)), 1e-12))

    krows = keys_ref[...]
    ktn = krows * (1.0 / jnp.maximum(
        jnp.sqrt(jnp.sum(krows * krows, axis=1, keepdims=True)), 1e-12))
    crows = ctx_ref[...]
    ctn = crows * (1.0 / jnp.maximum(
        jnp.sqrt(jnp.sum(crows * crows, axis=1, keepdims=True)), 1e-12))

    dims = (((1,), (1,)), ((), ()))
    s = lax.dot_general(qn, ktn, dims, preferred_element_type=jnp.float32)
    s = s + lax.dot_general(qcn, ctn, dims, preferred_element_type=jnp.float32)

    qt = s.shape[0]
    L = [ls_ref[:, i * chunk:(i + 1) * chunk] for i in range(TOPK)]
    I = [is_ref[:, i * chunk:(i + 1) * chunk] for i in range(TOPK)]
    lane = lax.broadcasted_iota(jnp.int32, (qt, chunk), 1)
    base = j * kt

    for ch in range(nch):
        idx = lane + (base + ch * chunk)
        c = s[:, ch * chunk:(ch + 1) * chunk]
        c = jnp.where(idx < ktotal, c, _NEG_INF)
        gt = [c > L[i] for i in range(TOPK)]
        newL = [jnp.where(gt[0], c, L[0])]
        newI = [jnp.where(gt[0], idx, I[0])]
        for i in range(1, TOPK):
            newL.append(jnp.where(gt[i - 1], L[i - 1],
                                  jnp.where(gt[i], c, L[i])))
            newI.append(jnp.where(gt[i - 1], I[i - 1],
                                  jnp.where(gt[i], idx, I[i])))
        L, I = newL, newI

    for i in range(TOPK):
        ls_ref[:, i * chunk:(i + 1) * chunk] = L[i]
        is_ref[:, i * chunk:(i + 1) * chunk] = I[i]

    @pl.when(j == nkt - 1)
    def _emit():
        cand = jnp.concatenate(L, axis=1)
        candi = jnp.concatenate(I, axis=1)
        for t in range(TOPK):
            m = jnp.max(cand, axis=1, keepdims=True)
            mi = jnp.min(jnp.where(cand == m, candi, _I32_MAX),
                         axis=1, keepdims=True)
            out_s_ref[:, t:t + 1] = m
            out_i_ref[:, t:t + 1] = mi
            cand = jnp.where((cand == m) & (candi == mi), _NEG_INF, cand)


def _score_topk(query_latent, query_context, keys, contexts, *,
                kt=2048, chunk=128):
    q, d = query_latent.shape
    k, _ = keys.shape
    p = query_context.shape[1]
    nkt = -(-k // kt)

    body = functools.partial(_score_topk_body, kt=kt, nkt=nkt, ktotal=k,
                             chunk=chunk)
    out_s, out_i = pl.pallas_call(
        body,
        grid=(nkt,),
        in_specs=[
            pl.BlockSpec((q, d), lambda j: (0, 0)),
            pl.BlockSpec((q, p), lambda j: (0, 0)),
            pl.BlockSpec((kt, d), lambda j: (j, 0)),
            pl.BlockSpec((kt, p), lambda j: (j, 0)),
        ],
        out_specs=[
            pl.BlockSpec((q, TOPK), lambda j: (0, 0)),
            pl.BlockSpec((q, TOPK), lambda j: (0, 0)),
        ],
        out_shape=[
            jax.ShapeDtypeStruct((q, TOPK), jnp.float32),
            jax.ShapeDtypeStruct((q, TOPK), jnp.int32),
        ],
        scratch_shapes=[
            pltpu.VMEM((q, TOPK * chunk), jnp.float32),
            pltpu.VMEM((q, TOPK * chunk), jnp.int32),
        ],
    )(query_latent, query_context, keys, contexts)
    return out_s, out_i


def kernel(query_latent, query_context, keys, contexts, ruls, sohs, k):
    topk_scores, topk_idx = _score_topk(query_latent, query_context, keys,
                                        contexts)
    retrieved_keys = jnp.take(keys, topk_idx, axis=0)
    retrieved_ruls = jnp.take(ruls, topk_idx, axis=0)
    retrieved_sohs = jnp.take(sohs, topk_idx, axis=0)
    return retrieved_keys, retrieved_ruls, retrieved_sohs, topk_scores
